# baseline (device time: 35821 ns/iter reference)
import jax
import jax.numpy as jnp
from jax import lax
from jax.experimental import pallas as pl
from jax.experimental.pallas import tpu as pltpu

N_DEV = 4
B, SQ, SKV, HQ, DH = 2, 512, 512, 8, 64
DM = 768
DQ = HQ * DH
ROWS = B * SQ
CH = ROWS // N_DEV


def kernel(x, Wq, K_ext, V_ext, Wo):
    my = lax.axis_index("i")
    Kt = lax.dynamic_slice_in_dim(K_ext, my * HQ, HQ, axis=2).reshape(
        B, SKV, DQ)
    Vt = lax.dynamic_slice_in_dim(V_ext, my * HQ, HQ, axis=2).reshape(
        B, SKV, DQ)
    x2 = x.reshape(ROWS, DM)

    def body(x_ref, wq_ref, kt_ref, vt_ref, wo_ref, out_ref,
             ctx_ref, part_ref, rs_ref, agsrc_ref, ag_ref,
             send_sems, rs_recv_sems, ag_send_sems, ag_recv_sems):
        me = lax.axis_index("i")

        barrier_sem = pltpu.get_barrier_semaphore()
        for r in range(1, N_DEV):
            pl.semaphore_signal(barrier_sem, inc=1,
                                device_id=(lax.rem(me + r, N_DEV),),
                                device_id_type=pl.DeviceIdType.MESH)
        pl.semaphore_wait(barrier_sem, N_DEV - 1)

        ki = lax.broadcasted_iota(jnp.int32, (CH, SKV), 1)
        qi0 = lax.broadcasted_iota(jnp.int32, (CH, SKV), 0)

        def compute_chunk(c):
            b = lax.div(c, 2)
            qoff = lax.rem(c, 2) * CH
            xq = x_ref[pl.ds(c * CH, CH), :]
            q_c = jnp.dot(xq, wq_ref[:],
                          preferred_element_type=jnp.float32) * 0.125
            qi = qi0 + qoff
            mask = (jnp.abs(qi - ki) <= 128) | (ki < 32) | (qi < 32)
            for h in range(HQ):
                q = q_c[:, h * DH:(h + 1) * DH]
                k = kt_ref[b, :, h * DH:(h + 1) * DH]
                s = lax.dot_general(q, k, (((1,), (1,)), ((), ())),
                                    preferred_element_type=jnp.float32)
                w = jnp.exp(jnp.where(mask, s, -1e9))
                ctx = jnp.dot(w, vt_ref[b, :, h * DH:(h + 1) * DH],
                              preferred_element_type=jnp.float32)
                ctx = ctx / jnp.sum(w, axis=1, keepdims=True)
                ctx_ref[:, h * DH:(h + 1) * DH] = ctx
            return jnp.dot(ctx_ref[:], wo_ref[:],
                           preferred_element_type=jnp.float32)

        HDM = DM // 2

        rs = [[], []]
        for r in range(1, N_DEV):
            c = lax.rem(me + r, N_DEV)
            part_ref[r - 1, :, :] = compute_chunk(c).astype(jnp.bfloat16)
            for hf in range(2):
                rdma = pltpu.make_async_remote_copy(
                    src_ref=part_ref.at[r - 1, :, pl.ds(hf * HDM, HDM)],
                    dst_ref=rs_ref.at[r - 1, :, pl.ds(hf * HDM, HDM)],
                    send_sem=send_sems.at[(r - 1) * 2 + hf],
                    recv_sem=rs_recv_sems.at[(r - 1) * 2 + hf],
                    device_id=(c,),
                    device_id_type=pl.DeviceIdType.MESH,
                )
                rdma.start()
                rs[hf].append(rdma)

        own = compute_chunk(me)

        ag = []
        for hf in range(2):
            for rdma in rs[hf]:
                rdma.wait_recv()
            cols = pl.ds(hf * HDM, HDM)
            redh = (own[:, hf * HDM:(hf + 1) * HDM]
                    + rs_ref[0, :, cols].astype(jnp.float32)
                    + rs_ref[1, :, cols].astype(jnp.float32)
                    + rs_ref[2, :, cols].astype(jnp.float32))
            out_ref[pl.ds(me * CH, CH), cols] = redh
            agsrc_ref[:, cols] = redh.astype(jnp.bfloat16)
            for r in range(1, N_DEV):
                p = lax.rem(me + r, N_DEV)
                rdma = pltpu.make_async_remote_copy(
                    src_ref=agsrc_ref.at[:, cols],
                    dst_ref=ag_ref.at[N_DEV - 1 - r, :, cols],
                    send_sem=ag_send_sems.at[(r - 1) * 2 + hf],
                    recv_sem=ag_recv_sems.at[(N_DEV - 1 - r) * 2 + hf],
                    device_id=(p,),
                    device_id_type=pl.DeviceIdType.MESH,
                )
                rdma.start()
                ag.append(rdma)

        for rdma in ag:
            rdma.wait_recv()
        for j in range(N_DEV - 1):
            p = lax.rem(me + j + 1, N_DEV)
            out_ref[pl.ds(p * CH, CH), :] = ag_ref[j].astype(jnp.float32)
        for hf in range(2):
            for rdma in rs[hf]:
                rdma.wait_send()
        for rdma in ag:
            rdma.wait_send()

    out = pl.pallas_call(
        body,
        out_shape=jax.ShapeDtypeStruct((ROWS, DM), jnp.float32),
        in_specs=[pl.BlockSpec(memory_space=pltpu.VMEM)] * 5,
        out_specs=pl.BlockSpec(memory_space=pltpu.VMEM),
        scratch_shapes=[
            pltpu.VMEM((CH, DQ), jnp.float32),
            pltpu.VMEM((N_DEV - 1, CH, DM), jnp.bfloat16),
            pltpu.VMEM((N_DEV - 1, CH, DM), jnp.bfloat16),
            pltpu.VMEM((CH, DM), jnp.bfloat16),
            pltpu.VMEM((N_DEV - 1, CH, DM), jnp.bfloat16),
            pltpu.SemaphoreType.DMA((2 * (N_DEV - 1),)),
            pltpu.SemaphoreType.DMA((2 * (N_DEV - 1),)),
            pltpu.SemaphoreType.DMA((2 * (N_DEV - 1),)),
            pltpu.SemaphoreType.DMA((2 * (N_DEV - 1),)),
        ],
        compiler_params=pltpu.CompilerParams(collective_id=0),
    )(x2, Wq, Kt, Vt, Wo)
    return out.reshape(B, SQ, DM)


# device time: 34166 ns/iter; 1.0484x vs baseline; 1.0484x over previous
import jax
import jax.numpy as jnp
from jax import lax
from jax.experimental import pallas as pl
from jax.experimental.pallas import tpu as pltpu

N_DEV = 4
B, SQ, SKV, HQ, DH = 2, 512, 512, 8, 64
DM = 768
DQ = HQ * DH
ROWS = B * SQ
CH = ROWS // N_DEV


def kernel(x, Wq, K_ext, V_ext, Wo):
    my = lax.axis_index("i")
    K = lax.dynamic_slice_in_dim(K_ext, my * HQ, HQ, axis=2)
    V = lax.dynamic_slice_in_dim(V_ext, my * HQ, HQ, axis=2)
    bf = jnp.bfloat16
    Kt = jnp.transpose(K.astype(bf), (0, 2, 1, 3)).reshape(B * HQ, SKV, DH)
    Vt = jnp.transpose(V.astype(bf), (0, 2, 1, 3)).reshape(B * HQ, SKV, DH)
    x2 = x.reshape(ROWS, DM)

    def body(x_ref, wq_ref, kt_ref, vt_ref, wo_ref, out_ref,
             ctx_ref, part_ref, rs_ref, agsrc_ref, ag_ref,
             send_sems, rs_recv_sems, ag_send_sems, ag_recv_sems):
        me = lax.axis_index("i")

        barrier_sem = pltpu.get_barrier_semaphore()
        for r in range(1, N_DEV):
            pl.semaphore_signal(barrier_sem, inc=1,
                                device_id=(lax.rem(me + r, N_DEV),),
                                device_id_type=pl.DeviceIdType.MESH)
        pl.semaphore_wait(barrier_sem, N_DEV - 1)

        ki = lax.broadcasted_iota(jnp.int32, (CH, SKV), 1)
        qi0 = lax.broadcasted_iota(jnp.int32, (CH, SKV), 0)

        def compute_chunk(c):
            b = lax.div(c, 2)
            qoff = lax.rem(c, 2) * CH
            xq = x_ref[pl.ds(c * CH, CH), :]
            q_c = jnp.dot(xq, wq_ref[:],
                          preferred_element_type=jnp.float32) * 0.125
            qi = qi0 + qoff
            mask = (jnp.abs(qi - ki) <= 128) | (ki < 32) | (qi < 32)
            for h in range(HQ):
                bh = b * HQ + h
                q = q_c[:, h * DH:(h + 1) * DH]
                k = kt_ref[bh].astype(jnp.float32)
                s = lax.dot_general(q, k, (((1,), (1,)), ((), ())),
                                    preferred_element_type=jnp.float32)
                w = jnp.exp(jnp.where(mask, s, -1e9))
                ctx = jnp.dot(w, vt_ref[bh].astype(jnp.float32),
                              preferred_element_type=jnp.float32)
                ctx = ctx / jnp.sum(w, axis=1, keepdims=True)
                ctx_ref[:, h * DH:(h + 1) * DH] = ctx
            return jnp.dot(ctx_ref[:], wo_ref[:],
                           preferred_element_type=jnp.float32)

        HDM = DM // 2

        rs = [[], []]
        for r in range(1, N_DEV):
            c = lax.rem(me + r, N_DEV)
            part_ref[r - 1, :, :] = compute_chunk(c).astype(jnp.bfloat16)
            for hf in range(2):
                rdma = pltpu.make_async_remote_copy(
                    src_ref=part_ref.at[r - 1, :, pl.ds(hf * HDM, HDM)],
                    dst_ref=rs_ref.at[r - 1, :, pl.ds(hf * HDM, HDM)],
                    send_sem=send_sems.at[(r - 1) * 2 + hf],
                    recv_sem=rs_recv_sems.at[(r - 1) * 2 + hf],
                    device_id=(c,),
                    device_id_type=pl.DeviceIdType.MESH,
                )
                rdma.start()
                rs[hf].append(rdma)

        own = compute_chunk(me)

        ag = []
        for hf in range(2):
            for rdma in rs[hf]:
                rdma.wait_recv()
            cols = pl.ds(hf * HDM, HDM)
            redh = (own[:, hf * HDM:(hf + 1) * HDM]
                    + rs_ref[0, :, cols].astype(jnp.float32)
                    + rs_ref[1, :, cols].astype(jnp.float32)
                    + rs_ref[2, :, cols].astype(jnp.float32))
            out_ref[pl.ds(me * CH, CH), cols] = redh
            agsrc_ref[:, cols] = redh.astype(jnp.bfloat16)
            for r in range(1, N_DEV):
                p = lax.rem(me + r, N_DEV)
                rdma = pltpu.make_async_remote_copy(
                    src_ref=agsrc_ref.at[:, cols],
                    dst_ref=ag_ref.at[N_DEV - 1 - r, :, cols],
                    send_sem=ag_send_sems.at[(r - 1) * 2 + hf],
                    recv_sem=ag_recv_sems.at[(N_DEV - 1 - r) * 2 + hf],
                    device_id=(p,),
                    device_id_type=pl.DeviceIdType.MESH,
                )
                rdma.start()
                ag.append(rdma)

        for rdma in ag:
            rdma.wait_recv()
        for j in range(N_DEV - 1):
            p = lax.rem(me + j + 1, N_DEV)
            out_ref[pl.ds(p * CH, CH), :] = ag_ref[j].astype(jnp.float32)
        for hf in range(2):
            for rdma in rs[hf]:
                rdma.wait_send()
        for rdma in ag:
            rdma.wait_send()

    out = pl.pallas_call(
        body,
        out_shape=jax.ShapeDtypeStruct((ROWS, DM), jnp.float32),
        in_specs=[pl.BlockSpec(memory_space=pltpu.VMEM)] * 5,
        out_specs=pl.BlockSpec(memory_space=pltpu.VMEM),
        scratch_shapes=[
            pltpu.VMEM((CH, DQ), jnp.float32),
            pltpu.VMEM((N_DEV - 1, CH, DM), jnp.bfloat16),
            pltpu.VMEM((N_DEV - 1, CH, DM), jnp.bfloat16),
            pltpu.VMEM((CH, DM), jnp.bfloat16),
            pltpu.VMEM((N_DEV - 1, CH, DM), jnp.bfloat16),
            pltpu.SemaphoreType.DMA((2 * (N_DEV - 1),)),
            pltpu.SemaphoreType.DMA((2 * (N_DEV - 1),)),
            pltpu.SemaphoreType.DMA((2 * (N_DEV - 1),)),
            pltpu.SemaphoreType.DMA((2 * (N_DEV - 1),)),
        ],
        compiler_params=pltpu.CompilerParams(collective_id=0),
    )(x2, Wq, Kt, Vt, Wo)
    return out.reshape(B, SQ, DM)
